# baseline (device time: 230410 ns/iter reference)
import jax
import jax.numpy as jnp
from jax import lax
from jax.experimental import pallas as pl
from jax.experimental.pallas import tpu as pltpu

N_DEV = 8


def _mlp_layer(x_shard, win, wout, *, collective_id):
    m, d = x_shard.shape
    dh = win.shape[1]

    def body(x_ref, win_ref, wout_ref, out_ref,
             xfull, part, rs_buf, ag_send, ag_recv, rs_send, rs_recv):
        my = lax.axis_index("i")
        left = lax.rem(my + N_DEV - 1, N_DEV)
        right = lax.rem(my + 1, N_DEV)

        barrier = pltpu.get_barrier_semaphore()
        for nbr in (left, right):
            pl.semaphore_signal(barrier, inc=1, device_id=(nbr,),
                                device_id_type=pl.DeviceIdType.MESH)
        pl.semaphore_wait(barrier, 2)

        xfull[pl.ds(my * m, m), :] = x_ref[...]

        for h in range(N_DEV - 1):
            blk = lax.rem(my - h + N_DEV, N_DEV)
            rdma = pltpu.make_async_remote_copy(
                src_ref=xfull.at[pl.ds(blk * m, m), :],
                dst_ref=xfull.at[pl.ds(blk * m, m), :],
                send_sem=ag_send.at[h],
                recv_sem=ag_recv.at[h],
                device_id=(right,),
                device_id_type=pl.DeviceIdType.MESH,
            )
            rdma.start()
            rdma.wait()

        hval = jnp.maximum(
            jnp.dot(xfull[...], win_ref[...],
                    preferred_element_type=jnp.float32),
            0.0,
        )
        part[...] = jnp.dot(hval, wout_ref[...],
                            preferred_element_type=jnp.float32)

        for s in range(N_DEV - 1):
            sblk = lax.rem(my + N_DEV - 1 - s, N_DEV)
            if s == 0:
                src = part.at[pl.ds(sblk * m, m), :]
            else:
                rs_buf[s - 1, :, :] = (
                    rs_buf[s - 1, :, :] + part[pl.ds(sblk * m, m), :]
                )
                src = rs_buf.at[s - 1]
            rdma = pltpu.make_async_remote_copy(
                src_ref=src,
                dst_ref=rs_buf.at[s],
                send_sem=rs_send.at[s],
                recv_sem=rs_recv.at[s],
                device_id=(right,),
                device_id_type=pl.DeviceIdType.MESH,
            )
            rdma.start()
            rdma.wait()

        out_ref[...] = rs_buf[N_DEV - 2] + part[pl.ds(my * m, m), :]

    return pl.pallas_call(
        body,
        out_shape=jax.ShapeDtypeStruct((m, d), jnp.float32),
        in_specs=[pl.BlockSpec(memory_space=pltpu.VMEM)] * 3,
        out_specs=pl.BlockSpec(memory_space=pltpu.VMEM),
        scratch_shapes=[
            pltpu.VMEM((N_DEV * m, d), jnp.float32),
            pltpu.VMEM((N_DEV * m, d), jnp.float32),
            pltpu.VMEM((N_DEV - 1, m, d), jnp.float32),
            pltpu.SemaphoreType.DMA((N_DEV - 1,)),
            pltpu.SemaphoreType.DMA((N_DEV - 1,)),
            pltpu.SemaphoreType.DMA((N_DEV - 1,)),
            pltpu.SemaphoreType.DMA((N_DEV - 1,)),
        ],
        compiler_params=pltpu.CompilerParams(collective_id=collective_id),
    )(x_shard, win, wout)


def kernel(x, Win0, Wout0, Win1, Wout1, Win2, Wout2):
    x = _mlp_layer(x, Win0, Wout0, collective_id=0)
    x = _mlp_layer(x, Win1, Wout1, collective_id=1)
    x = _mlp_layer(x, Win2, Wout2, collective_id=2)
    return x


# device time: 123317 ns/iter; 1.8684x vs baseline; 1.8684x over previous
import jax
import jax.numpy as jnp
from jax import lax
from jax.experimental import pallas as pl
from jax.experimental.pallas import tpu as pltpu

N_DEV = 8
R_HOPS = N_DEV // 2
L_HOPS = N_DEV - 1 - R_HOPS
N_STEP = N_DEV - 1


def _mlp_layer(x_shard, win, wout, *, collective_id):
    m, d = x_shard.shape
    half = d // 2

    def body(x_ref, win_ref, wout_ref, out_ref,
             xfull, rsr_buf, rsl_buf,
             agr_send, agr_recv, agl_send, agl_recv,
             rsr_send, rsr_recv, rsl_send, rsl_recv):
        my = lax.axis_index("i")
        left = lax.rem(my + N_DEV - 1, N_DEV)
        right = lax.rem(my + 1, N_DEV)

        def blk_ds(delta):
            blk = lax.rem(my + delta + N_DEV, N_DEV)
            return pl.ds(blk * m, m)

        barrier = pltpu.get_barrier_semaphore()
        for nbr in (left, right):
            pl.semaphore_signal(barrier, inc=1, device_id=(nbr,),
                                device_id_type=pl.DeviceIdType.MESH)
        pl.semaphore_wait(barrier, 2)

        def ag_rdma(delta, dst_dev, send_sem, recv_sem):
            src = x_ref if delta == 0 else xfull.at[blk_ds(delta), :]
            return pltpu.make_async_remote_copy(
                src_ref=src,
                dst_ref=xfull.at[blk_ds(delta), :],
                send_sem=send_sem, recv_sem=recv_sem,
                device_id=(dst_dev,), device_id_type=pl.DeviceIdType.MESH,
            )

        agr = [ag_rdma(-r, right, agr_send.at[r], agr_recv.at[r])
               for r in range(R_HOPS)]
        agl = [ag_rdma(+l, left, agl_send.at[l], agl_recv.at[l])
               for l in range(L_HOPS)]

        agr[0].start()
        agl[0].start()

        waited = set()

        def wait_ag(lane, idx):
            if (lane, idx) not in waited:
                (agr if lane == "r" else agl)[idx].wait_recv()
                waited.add((lane, idx))

        pv = {}

        def ensure(delta):
            key = delta % N_DEV
            if key in pv:
                return pv[key]
            if delta == 0:
                xb = x_ref[...]
            elif delta < 0:
                wait_ag("r", -delta - 1)
                xb = xfull[blk_ds(delta), :]
            else:
                wait_ag("l", delta - 1)
                xb = xfull[blk_ds(delta), :]
            h = jnp.maximum(
                jnp.dot(xb, win_ref[...], preferred_element_type=jnp.float32),
                0.0)
            pv[key] = jnp.dot(h, wout_ref[...],
                              preferred_element_type=jnp.float32)
            return pv[key]

        pv0 = ensure(0)

        rsr = []
        rsl = []
        for s in range(N_STEP):
            if s < R_HOPS:
                wait_ag("r", s)
                if s < R_HOPS - 1:
                    agr[s + 1].start()
            if s < L_HOPS:
                wait_ag("l", s)
                if s < L_HOPS - 1:
                    agl[s + 1].start()

            dr = -(s + 1) if s < R_HOPS else N_DEV - 1 - s
            p = ensure(dr)
            if s == 0:
                rsr_buf[N_STEP, :, :] = p[:, :half]
                src = rsr_buf.at[N_STEP]
            else:
                rsr[s - 1].wait_recv()
                rsr_buf[s - 1, :, :] = rsr_buf[s - 1, :, :] + p[:, :half]
                src = rsr_buf.at[s - 1]
            rdma = pltpu.make_async_remote_copy(
                src_ref=src, dst_ref=rsr_buf.at[s],
                send_sem=rsr_send.at[s], recv_sem=rsr_recv.at[s],
                device_id=(right,), device_id_type=pl.DeviceIdType.MESH,
            )
            rdma.start()
            rsr.append(rdma)

            dl = (s + 1) if s < R_HOPS else s + 1 - N_DEV
            p = ensure(dl)
            if s == 0:
                rsl_buf[N_STEP, :, :] = p[:, half:]
                src = rsl_buf.at[N_STEP]
            else:
                rsl[s - 1].wait_recv()
                rsl_buf[s - 1, :, :] = rsl_buf[s - 1, :, :] + p[:, half:]
                src = rsl_buf.at[s - 1]
            rdma = pltpu.make_async_remote_copy(
                src_ref=src, dst_ref=rsl_buf.at[s],
                send_sem=rsl_send.at[s], recv_sem=rsl_recv.at[s],
                device_id=(left,), device_id_type=pl.DeviceIdType.MESH,
            )
            rdma.start()
            rsl.append(rdma)

        rsr[N_STEP - 1].wait_recv()
        rsl[N_STEP - 1].wait_recv()
        out_ref[:, :half] = rsr_buf[N_STEP - 1] + pv0[:, :half]
        out_ref[:, half:] = rsl_buf[N_STEP - 1] + pv0[:, half:]

        for rd in agr + agl + rsr + rsl:
            rd.wait_send()

    return pl.pallas_call(
        body,
        out_shape=jax.ShapeDtypeStruct((m, d), jnp.float32),
        in_specs=[pl.BlockSpec(memory_space=pltpu.VMEM)] * 3,
        out_specs=pl.BlockSpec(memory_space=pltpu.VMEM),
        scratch_shapes=[
            pltpu.VMEM((N_DEV * m, d), jnp.float32),
            pltpu.VMEM((N_STEP + 1, m, half), jnp.float32),
            pltpu.VMEM((N_STEP + 1, m, half), jnp.float32),
            pltpu.SemaphoreType.DMA((R_HOPS,)),
            pltpu.SemaphoreType.DMA((R_HOPS,)),
            pltpu.SemaphoreType.DMA((L_HOPS,)),
            pltpu.SemaphoreType.DMA((L_HOPS,)),
            pltpu.SemaphoreType.DMA((N_STEP,)),
            pltpu.SemaphoreType.DMA((N_STEP,)),
            pltpu.SemaphoreType.DMA((N_STEP,)),
            pltpu.SemaphoreType.DMA((N_STEP,)),
        ],
        compiler_params=pltpu.CompilerParams(collective_id=collective_id),
    )(x_shard, win, wout)


def kernel(x, Win0, Wout0, Win1, Wout1, Win2, Wout2):
    x = _mlp_layer(x, Win0, Wout0, collective_id=0)
    x = _mlp_layer(x, Win1, Wout1, collective_id=1)
    x = _mlp_layer(x, Win2, Wout2, collective_id=2)
    return x
